# baseline (speedup 1.0000x reference)
import jax
import jax.numpy as jnp
from jax import lax
from jax.experimental import pallas as pl
from jax.experimental.pallas import tpu as pltpu
from jax.experimental.pallas import tpu_sc as plsc

N = 10000
D = 128
G = 64
NC = 2
NS = 16
NW = NC * NS
CH = 128
NCH0 = 40
NCH1 = 120
RPW = 640
NPAD = NS * RPW
ZR = 40
CR = 32

_f32 = jnp.float32


def _make_agg_body():
    def _agg_body(h_hbm, src_hbm, dst_hbm, out_hbm,
                  idx_dst, srcb, rows, agg_sh, sem_i, sem_g):
        c = lax.axis_index("c")
        s = lax.axis_index("s")
        zero16 = jnp.zeros((16,), _f32)

        def zrow(r, _):
            def zcol(k, _):
                rows[0, r, pl.ds(k * 16, 16)] = zero16
                return 0
            return lax.fori_loop(0, D // 16, zcol, 0)
        lax.fori_loop(0, CH, zrow, 0)

        def zcopy(t, _):
            pltpu.sync_copy(rows.at[0], agg_sh.at[pl.ds(s * RPW + t * CH, CH)])
            return 0
        lax.fori_loop(0, RPW // CH, zcopy, 0)

        pltpu.sync_copy(dst_hbm.at[c, s], idx_dst)
        pltpu.sync_copy(src_hbm.at[c, s, 0], srcb.at[0])
        plsc.subcore_barrier()

        nch = jnp.where(c == 0, NCH0, NCH1).astype(jnp.int32)

        pltpu.async_copy(h_hbm.at[srcb.at[0]], rows.at[0], sem_g)

        @pl.when(nch > 1)
        def _():
            pltpu.async_copy(src_hbm.at[c, s, 1], srcb.at[1], sem_i)

        def edge(j, _):
            slot = lax.rem(j, 2)
            pltpu.make_async_copy(h_hbm.at[srcb.at[slot]], rows.at[slot],
                                  sem_g).wait()

            @pl.when(j + 1 < nch)
            def _():
                nslot = 1 - slot
                pltpu.make_async_copy(src_hbm.at[c, s, j + 1],
                                      srcb.at[nslot], sem_i).wait()
                pltpu.async_copy(h_hbm.at[srcb.at[nslot]],
                                 rows.at[nslot], sem_g)

                @pl.when(j + 2 < nch)
                def _():
                    pltpu.async_copy(src_hbm.at[c, s, j + 2],
                                     srcb.at[slot], sem_i)
            pltpu.sync_copy(rows.at[slot], agg_sh.at[idx_dst.at[j]], add=True)
            return 0
        lax.fori_loop(0, nch, edge, 0)

        plsc.subcore_barrier()
        pltpu.sync_copy(agg_sh.at[pl.ds(s * RPW, RPW)],
                        out_hbm.at[c, pl.ds(s * RPW, RPW)])
    return _agg_body


def _agg(h, src4, dst4):
    mesh = plsc.VectorSubcoreMesh(core_axis_name="c", subcore_axis_name="s")
    nch_max = max(NCH0, NCH1)
    return pl.kernel(
        _make_agg_body(),
        out_type=jax.ShapeDtypeStruct((NC, NPAD, D), _f32),
        mesh=mesh,
        scratch_types=[
            pltpu.VMEM((nch_max, CH), jnp.int32),
            pltpu.VMEM((2, CH), jnp.int32),
            pltpu.VMEM((2, CH, D), _f32),
            pltpu.VMEM_SHARED((NPAD, D), _f32),
            pltpu.SemaphoreType.DMA,
            pltpu.SemaphoreType.DMA,
        ],
    )(h, src4, dst4)


def _bdot(a, b):
    return jnp.dot(a.astype(jnp.bfloat16), b.astype(jnp.bfloat16),
                   preferred_element_type=_f32)


def _mlp_body(h_ref, a0_ref, a1_ref, w1_ref, b1_ref, w2_ref, b2_ref, o_ref):
    m = h_ref[...] + (a0_ref[0] + a1_ref[0])
    t = jnp.maximum(_bdot(m, w1_ref[...]) + b1_ref[...], 0.0)
    o = _bdot(t, w2_ref[...]) + b2_ref[...]
    o_ref[...] = jnp.maximum(o, 0.0)


def _mlp(h, agg, w1, b1, w2, b2):
    bm = 2000
    return pl.pallas_call(
        _mlp_body,
        grid=(N // bm,),
        in_specs=[
            pl.BlockSpec((bm, D), lambda i: (i, 0)),
            pl.BlockSpec((1, bm, D), lambda i: (0, i, 0)),
            pl.BlockSpec((1, bm, D), lambda i: (1, i, 0)),
            pl.BlockSpec((D, D), lambda i: (0, 0)),
            pl.BlockSpec((1, D), lambda i: (0, 0)),
            pl.BlockSpec((D, D), lambda i: (0, 0)),
            pl.BlockSpec((1, D), lambda i: (0, 0)),
        ],
        out_specs=pl.BlockSpec((bm, D), lambda i: (i, 0)),
        out_shape=jax.ShapeDtypeStruct((N, D), _f32),
    )(h, agg, agg, w1, b1.reshape(1, D), w2, b2.reshape(1, D))


def _lanesum(a):
    iota = lax.iota(jnp.int32, 16)
    for sh in (8, 4, 2, 1):
        a = a + a[(iota + sh) % 16]
    return a[0]


def _bounds(batch_v, g):
    z = jnp.zeros((16,), jnp.int32)

    one = jnp.ones((16,), jnp.int32)
    zero = jnp.zeros((16,), jnp.int32)

    def body(i, acc):
        a0, a1, a2 = acc
        v = batch_v[pl.ds(i * 16, 16)]
        a0 = a0 + jnp.minimum(jnp.maximum(g - v, zero), one)
        a1 = a1 + jnp.minimum(jnp.maximum(g + 1 - v, zero), one)
        a2 = a2 + jnp.minimum(jnp.maximum(g + 2 - v, zero), one)
        return (a0, a1, a2)

    a0, a1, a2 = lax.fori_loop(0, N // 16, body, (z, z, z))
    return _lanesum(a0), _lanesum(a1), _lanesum(a2)


def _pool_body(h_hbm, batch_hbm, x_hbm, pooled_hbm, xroot_hbm,
               batch_v, idxb, chunk, obuf, xbuf, sem):
    c = lax.axis_index("c")
    s = lax.axis_index("s")
    wid = s * NC + c
    pltpu.sync_copy(batch_hbm, batch_v)
    neg = jnp.full((16,), -jnp.inf, _f32)
    iota = lax.iota(jnp.int32, 16)
    b0, b1, b2 = _bounds(batch_v, wid * 2)
    for gi in range(2):
        start = (b0, b1)[gi]
        end = (b1, b2)[gi]
        last = end - 1
        cnt = end - start
        nch = (cnt + CR - 1) // CR

        def fbody(t, acc):
            base = start + t * CR
            for k in range(CR // 16):
                idxb[pl.ds(k * 16, 16)] = jnp.minimum(base + k * 16 + iota,
                                                      last)
            pltpu.async_copy(h_hbm.at[idxb], chunk, sem).wait()
            accl = list(acc)
            for r in range(CR):
                for k in range(D // 16):
                    accl[k] = jnp.maximum(accl[k], chunk[r, pl.ds(k * 16, 16)])
            return tuple(accl)

        acc = lax.fori_loop(0, nch, fbody, (neg,) * (D // 16))
        for k in range(D // 16):
            obuf[gi, pl.ds(k * 16, 16)] = acc[k]
    lane0 = jnp.maximum(1 - iota, 0)
    idxb[pl.ds(0, 16)] = b1 + (b0 - b1) * lane0
    pltpu.async_copy(x_hbm.at[idxb.at[pl.ds(0, 16)]],
                     xbuf, sem).wait()
    pltpu.sync_copy(obuf, pooled_hbm.at[wid])
    pltpu.sync_copy(xbuf.at[pl.ds(0, 2)], xroot_hbm.at[wid])


def _pool(h, batch, x):
    mesh = plsc.VectorSubcoreMesh(core_axis_name="c", subcore_axis_name="s")
    return pl.kernel(
        _pool_body,
        out_type=(jax.ShapeDtypeStruct((NW, 2, D), _f32),
                  jax.ShapeDtypeStruct((NW, 2, D), _f32)),
        mesh=mesh,
        scratch_types=[
            pltpu.VMEM((N,), jnp.int32),
            pltpu.VMEM((CR,), jnp.int32),
            pltpu.VMEM((CR, D), _f32),
            pltpu.VMEM((2, D), _f32),
            pltpu.VMEM((16, D), _f32),
            pltpu.SemaphoreType.DMA,
        ],
    )(h, batch, x)


def _head_body(p_ref, xr_ref, l2w_ref, l2b_ref, lnw_ref, lnb_ref,
               l3w_ref, l3b_ref, o_ref):
    h2 = jnp.maximum(_bdot(p_ref[...], l2w_ref[...]) + l2b_ref[...], 0.0)
    news = jnp.maximum(_bdot(xr_ref[...], lnw_ref[...]) + lnb_ref[...], 0.0)
    z = _bdot(h2, l3w_ref[0]) + _bdot(news, l3w_ref[1]) + l3b_ref[...]
    o_ref[...] = jax.nn.sigmoid(z)


def _head(pooled, xroot, l2w, l2b, lnw, lnb, l3w, l3b):
    out = l3w.shape[1]
    return pl.pallas_call(
        _head_body,
        out_shape=jax.ShapeDtypeStruct((G, out), _f32),
    )(pooled, xroot, l2w, l2b.reshape(1, D), lnw, lnb.reshape(1, D),
      l3w.reshape(2, D, out), l3b.reshape(1, out))


def kernel(x, adj, batch, W1_0, b1_0, W2_0, b2_0, W1_1, b1_1, W2_1, b2_1,
           W1_2, b1_2, W2_2, b2_2, linnews_W, linnews_b, lin2_W, lin2_b,
           lin3_W, lin3_b):
    e = adj.shape[1]
    e0 = NS * NCH0 * CH
    e1 = NS * NCH1 * CH
    pad = e0 + e1 - e
    nch_max = max(NCH0, NCH1)
    src = jnp.concatenate([adj[0], jnp.zeros((pad,), jnp.int32)])
    dst = jnp.concatenate([adj[1], jnp.full((pad,), N, jnp.int32)])

    def core_view(a, fill):
        a0 = a[:e0].reshape(1, NS, NCH0, CH)
        a0 = jnp.concatenate(
            [a0, jnp.full((1, NS, nch_max - NCH0, CH), fill, jnp.int32)], axis=2)
        a1 = a[e0:].reshape(1, NS, NCH1, CH)
        a1 = jnp.concatenate(
            [a1, jnp.full((1, NS, nch_max - NCH1, CH), fill, jnp.int32)], axis=2)
        return jnp.concatenate([a0, a1], axis=0)

    src4 = core_view(src, 0)
    dst4 = core_view(dst, N)

    h = x
    layers = [(W1_0, b1_0, W2_0, b2_0), (W1_1, b1_1, W2_1, b2_1),
              (W1_2, b1_2, W2_2, b2_2)]
    for w1, b1, w2, b2 in layers:
        agg = _agg(h, src4, dst4)
        h = _mlp(h, agg, w1, b1, w2, b2)

    pooled, xroot = _pool(h, batch, x)
    return _head(pooled.reshape(G, D), xroot.reshape(G, D),
                 lin2_W, lin2_b, linnews_W, linnews_b, lin3_W, lin3_b)


# speedup vs baseline: 1.1668x; 1.1668x over previous
import jax
import jax.numpy as jnp
from jax import lax
from jax.experimental import pallas as pl
from jax.experimental.pallas import tpu as pltpu
from jax.experimental.pallas import tpu_sc as plsc

N = 10000
D = 128
G = 64
NC = 2
NS = 16
NW = NC * NS
CH = 128
NCH0 = 120
NCH1 = 40
RPW = 640
NPAD = NS * RPW
ZR = 40
CR = 32

_f32 = jnp.float32


def _make_agg_body():
    def _agg_body(h_hbm, src_hbm, dst_hbm, out_hbm,
                  idx_dst, srcb, rows, agg_sh, sem_i, sem_g):
        c = lax.axis_index("c")
        s = lax.axis_index("s")
        zero16 = jnp.zeros((16,), _f32)

        def zrow(r, _):
            def zcol(k, _):
                rows[0, r, pl.ds(k * 16, 16)] = zero16
                return 0
            return lax.fori_loop(0, D // 16, zcol, 0)
        lax.fori_loop(0, CH, zrow, 0)

        def zcopy(t, _):
            pltpu.sync_copy(rows.at[0], agg_sh.at[pl.ds(s * RPW + t * CH, CH)])
            return 0
        lax.fori_loop(0, RPW // CH, zcopy, 0)

        pltpu.sync_copy(dst_hbm.at[c, s], idx_dst)
        pltpu.sync_copy(src_hbm.at[c, s, 0], srcb.at[0])
        plsc.subcore_barrier()

        nch = jnp.where(c == 0, NCH0, NCH1).astype(jnp.int32)

        pltpu.async_copy(h_hbm.at[srcb.at[0]], rows.at[0], sem_g)

        @pl.when(nch > 1)
        def _():
            pltpu.async_copy(src_hbm.at[c, s, 1], srcb.at[1], sem_i)

        def edge(j, _):
            slot = lax.rem(j, 2)
            pltpu.make_async_copy(h_hbm.at[srcb.at[slot]], rows.at[slot],
                                  sem_g).wait()

            @pl.when(j + 1 < nch)
            def _():
                nslot = 1 - slot
                pltpu.make_async_copy(src_hbm.at[c, s, j + 1],
                                      srcb.at[nslot], sem_i).wait()
                pltpu.async_copy(h_hbm.at[srcb.at[nslot]],
                                 rows.at[nslot], sem_g)

                @pl.when(j + 2 < nch)
                def _():
                    pltpu.async_copy(src_hbm.at[c, s, j + 2],
                                     srcb.at[slot], sem_i)
            pltpu.sync_copy(rows.at[slot], agg_sh.at[idx_dst.at[j]], add=True)
            return 0
        lax.fori_loop(0, nch, edge, 0)

        plsc.subcore_barrier()
        pltpu.sync_copy(agg_sh.at[pl.ds(s * RPW, RPW)],
                        out_hbm.at[c, pl.ds(s * RPW, RPW)])
    return _agg_body


def _agg(h, src4, dst4):
    mesh = plsc.VectorSubcoreMesh(core_axis_name="c", subcore_axis_name="s")
    nch_max = max(NCH0, NCH1)
    return pl.kernel(
        _make_agg_body(),
        out_type=jax.ShapeDtypeStruct((NC, NPAD, D), _f32),
        mesh=mesh,
        scratch_types=[
            pltpu.VMEM((nch_max, CH), jnp.int32),
            pltpu.VMEM((2, CH), jnp.int32),
            pltpu.VMEM((2, CH, D), _f32),
            pltpu.VMEM_SHARED((NPAD, D), _f32),
            pltpu.SemaphoreType.DMA,
            pltpu.SemaphoreType.DMA,
        ],
    )(h, src4, dst4)


def _bdot(a, b):
    return jnp.dot(a.astype(jnp.bfloat16), b.astype(jnp.bfloat16),
                   preferred_element_type=_f32)


def _mlp_body(h_ref, a0_ref, a1_ref, w1_ref, b1_ref, w2_ref, b2_ref, o_ref):
    m = h_ref[...] + (a0_ref[0] + a1_ref[0])
    t = jnp.maximum(_bdot(m, w1_ref[...]) + b1_ref[...], 0.0)
    o = _bdot(t, w2_ref[...]) + b2_ref[...]
    o_ref[...] = jnp.maximum(o, 0.0)


def _mlp(h, agg, w1, b1, w2, b2):
    bm = 2000
    return pl.pallas_call(
        _mlp_body,
        grid=(N // bm,),
        in_specs=[
            pl.BlockSpec((bm, D), lambda i: (i, 0)),
            pl.BlockSpec((1, bm, D), lambda i: (0, i, 0)),
            pl.BlockSpec((1, bm, D), lambda i: (1, i, 0)),
            pl.BlockSpec((D, D), lambda i: (0, 0)),
            pl.BlockSpec((1, D), lambda i: (0, 0)),
            pl.BlockSpec((D, D), lambda i: (0, 0)),
            pl.BlockSpec((1, D), lambda i: (0, 0)),
        ],
        out_specs=pl.BlockSpec((bm, D), lambda i: (i, 0)),
        out_shape=jax.ShapeDtypeStruct((N, D), _f32),
    )(h, agg, agg, w1, b1.reshape(1, D), w2, b2.reshape(1, D))


def _lanesum(a):
    iota = lax.iota(jnp.int32, 16)
    for sh in (8, 4, 2, 1):
        a = a + a[(iota + sh) % 16]
    return a[0]


def _bounds(batch_v, g):
    z = jnp.zeros((16,), jnp.int32)

    one = jnp.ones((16,), jnp.int32)
    zero = jnp.zeros((16,), jnp.int32)

    def body(i, acc):
        a0, a1, a2 = acc
        v = batch_v[pl.ds(i * 16, 16)]
        a0 = a0 + jnp.minimum(jnp.maximum(g - v, zero), one)
        a1 = a1 + jnp.minimum(jnp.maximum(g + 1 - v, zero), one)
        a2 = a2 + jnp.minimum(jnp.maximum(g + 2 - v, zero), one)
        return (a0, a1, a2)

    a0, a1, a2 = lax.fori_loop(0, N // 16, body, (z, z, z))
    return _lanesum(a0), _lanesum(a1), _lanesum(a2)


def _pool_body(h_hbm, batch_hbm, x_hbm, pooled_hbm, xroot_hbm,
               batch_v, idxb, chunk, obuf, xbuf, sem):
    c = lax.axis_index("c")
    s = lax.axis_index("s")
    wid = s * NC + c
    pltpu.sync_copy(batch_hbm, batch_v)
    neg = jnp.full((16,), -jnp.inf, _f32)
    iota = lax.iota(jnp.int32, 16)
    b0, b1, b2 = _bounds(batch_v, wid * 2)
    for gi in range(2):
        start = (b0, b1)[gi]
        end = (b1, b2)[gi]
        last = end - 1
        cnt = end - start
        nch = (cnt + CR - 1) // CR

        def fbody(t, acc):
            base = start + t * CR
            for k in range(CR // 16):
                idxb[pl.ds(k * 16, 16)] = jnp.minimum(base + k * 16 + iota,
                                                      last)
            pltpu.async_copy(h_hbm.at[idxb], chunk, sem).wait()
            accl = list(acc)
            for r in range(CR):
                for k in range(D // 16):
                    accl[k] = jnp.maximum(accl[k], chunk[r, pl.ds(k * 16, 16)])
            return tuple(accl)

        acc = lax.fori_loop(0, nch, fbody, (neg,) * (D // 16))
        for k in range(D // 16):
            obuf[gi, pl.ds(k * 16, 16)] = acc[k]
    lane0 = jnp.maximum(1 - iota, 0)
    idxb[pl.ds(0, 16)] = b1 + (b0 - b1) * lane0
    pltpu.async_copy(x_hbm.at[idxb.at[pl.ds(0, 16)]],
                     xbuf, sem).wait()
    pltpu.sync_copy(obuf, pooled_hbm.at[wid])
    pltpu.sync_copy(xbuf.at[pl.ds(0, 2)], xroot_hbm.at[wid])


def _pool(h, batch, x):
    mesh = plsc.VectorSubcoreMesh(core_axis_name="c", subcore_axis_name="s")
    return pl.kernel(
        _pool_body,
        out_type=(jax.ShapeDtypeStruct((NW, 2, D), _f32),
                  jax.ShapeDtypeStruct((NW, 2, D), _f32)),
        mesh=mesh,
        scratch_types=[
            pltpu.VMEM((N,), jnp.int32),
            pltpu.VMEM((CR,), jnp.int32),
            pltpu.VMEM((CR, D), _f32),
            pltpu.VMEM((2, D), _f32),
            pltpu.VMEM((16, D), _f32),
            pltpu.SemaphoreType.DMA,
        ],
    )(h, batch, x)


def _head_body(p_ref, xr_ref, l2w_ref, l2b_ref, lnw_ref, lnb_ref,
               l3w_ref, l3b_ref, o_ref):
    h2 = jnp.maximum(_bdot(p_ref[...], l2w_ref[...]) + l2b_ref[...], 0.0)
    news = jnp.maximum(_bdot(xr_ref[...], lnw_ref[...]) + lnb_ref[...], 0.0)
    z = _bdot(h2, l3w_ref[0]) + _bdot(news, l3w_ref[1]) + l3b_ref[...]
    o_ref[...] = jax.nn.sigmoid(z)


def _head(pooled, xroot, l2w, l2b, lnw, lnb, l3w, l3b):
    out = l3w.shape[1]
    return pl.pallas_call(
        _head_body,
        out_shape=jax.ShapeDtypeStruct((G, out), _f32),
    )(pooled, xroot, l2w, l2b.reshape(1, D), lnw, lnb.reshape(1, D),
      l3w.reshape(2, D, out), l3b.reshape(1, out))


def kernel(x, adj, batch, W1_0, b1_0, W2_0, b2_0, W1_1, b1_1, W2_1, b2_1,
           W1_2, b1_2, W2_2, b2_2, linnews_W, linnews_b, lin2_W, lin2_b,
           lin3_W, lin3_b):
    e = adj.shape[1]
    e0 = NS * NCH0 * CH
    e1 = NS * NCH1 * CH
    pad = e0 + e1 - e
    nch_max = max(NCH0, NCH1)
    src = jnp.concatenate([adj[0], jnp.zeros((pad,), jnp.int32)])
    dst = jnp.concatenate([adj[1], jnp.full((pad,), N, jnp.int32)])

    def core_view(a, fill):
        a0 = a[:e0].reshape(1, NS, NCH0, CH)
        a0 = jnp.concatenate(
            [a0, jnp.full((1, NS, nch_max - NCH0, CH), fill, jnp.int32)], axis=2)
        a1 = a[e0:].reshape(1, NS, NCH1, CH)
        a1 = jnp.concatenate(
            [a1, jnp.full((1, NS, nch_max - NCH1, CH), fill, jnp.int32)], axis=2)
        return jnp.concatenate([a0, a1], axis=0)

    src4 = core_view(src, 0)
    dst4 = core_view(dst, N)

    h = x
    layers = [(W1_0, b1_0, W2_0, b2_0), (W1_1, b1_1, W2_1, b2_1),
              (W1_2, b1_2, W2_2, b2_2)]
    for w1, b1, w2, b2 in layers:
        agg = _agg(h, src4, dst4)
        h = _mlp(h, agg, w1, b1, w2, b2)

    pooled, xroot = _pool(h, batch, x)
    return _head(pooled.reshape(G, D), xroot.reshape(G, D),
                 lin2_W, lin2_b, linnews_W, linnews_b, lin3_W, lin3_b)
